# baseline (speedup 1.0000x reference)
import functools

import jax
import jax.numpy as jnp
from jax import lax
from jax.experimental import pallas as pl
from jax.experimental.pallas import tpu as pltpu
from jax.experimental.pallas import tpu_sc as plsc

D_MODEL = 1024
N_EXP = 8
N_TOK = 4096
D_FF = 4096
N_ASSIGN = 2 * N_TOK
TILE = 256
N_TILES = N_ASSIGN // TILE
N_ITEMS = 40
FF_CHUNK = 2048
N_FF = D_FF // FF_CHUNK

_f32 = jnp.float32
_i32 = jnp.int32


def _tr(v, n):
    r = lax.broadcasted_iota(_i32, (n, n), 0)
    c = lax.broadcasted_iota(_i32, (n, n), 1)
    ident = (r == c).astype(v.dtype)
    return lax.dot_general(ident, v, (((1,), (1,)), ((), ())),
                           preferred_element_type=v.dtype,
                           precision=lax.Precision.HIGHEST)


def _route_body(x_ref, wg_ref, bg_ref, pos_ref, w01_ref, items_ref):
    x = x_ref[...]
    wg = wg_ref[...]
    logits = lax.dot_general(x, wg, (((1,), (1,)), ((), ())),
                             preferred_element_type=_f32)
    logits = logits + bg_ref[...][0][None, :]

    m = jnp.max(logits, axis=1, keepdims=True)
    ex = jnp.exp(logits - m)
    w = ex / jnp.sum(ex, axis=1, keepdims=True)

    r8 = lax.broadcasted_iota(_i32, (N_EXP, N_EXP), 0)
    c8 = lax.broadcasted_iota(_i32, (N_EXP, N_EXP), 1)
    tri8 = (r8 <= c8).astype(_f32)
    m1 = jnp.max(logits, axis=1, keepdims=True)
    match1 = (logits == m1).astype(_f32)
    cum1 = lax.dot_general(match1, tri8, (((1,), (0,)), ((), ())),
                           preferred_element_type=_f32)
    first1 = match1 * (cum1 == 1.0).astype(_f32)
    l2 = jnp.where(first1 > 0, -jnp.inf, logits)
    m2 = jnp.max(l2, axis=1, keepdims=True)
    match2 = (l2 == m2).astype(_f32)
    cum2 = lax.dot_general(match2, tri8, (((1,), (0,)), ((), ())),
                           preferred_element_type=_f32)
    first2 = match2 * (cum2 == 1.0).astype(_f32)

    w0 = jnp.sum(first1 * w, axis=1)
    w1 = jnp.sum(first2 * w, axis=1)
    w01_ref[...] = jnp.stack([w0, w1], axis=0)

    oh = jnp.concatenate([first1, first2], axis=0)
    ch = 1024
    rch = lax.broadcasted_iota(_i32, (ch, ch), 0)
    cch = lax.broadcasted_iota(_i32, (ch, ch), 1)
    tri = (rch >= cch).astype(_f32)
    carry = jnp.zeros((1, N_EXP), _f32)
    cums = []
    for i in range(N_ASSIGN // ch):
        blk = oh[i * ch:(i + 1) * ch, :]
        cums.append(lax.dot_general(tri, blk, (((1,), (0,)), ((), ())),
                                    preferred_element_type=_f32) + carry)
        carry = carry + jnp.sum(blk, axis=0, keepdims=True)
    cum = jnp.concatenate(cums, axis=0)
    counts = carry

    ru = lax.broadcasted_iota(_i32, (N_EXP, N_EXP), 0)
    cu = lax.broadcasted_iota(_i32, (N_EXP, N_EXP), 1)
    strict = (ru < cu).astype(_f32)
    offs = lax.dot_general(counts, strict, (((1,), (0,)), ((), ())),
                           preferred_element_type=_f32,
                           precision=lax.Precision.HIGHEST)

    posf = jnp.sum(oh * (offs + cum - 1.0), axis=1)
    pos_ref[...] = posf.astype(_i32).reshape(2, N_TOK)

    offs9 = jnp.concatenate(
        [offs, jnp.full((1, 1), float(N_ASSIGN), _f32)], axis=1)
    offs9c = _tr(offs9, 9)

    iota32 = lax.broadcasted_iota(_i32, (1, N_TILES), 1).astype(_f32)
    pb = 256.0 * iota32
    le = (offs9c[0:8, :] <= pb).astype(_f32)
    e_base = jnp.sum(le, axis=0, keepdims=True) - 1.0
    seg_hi = jnp.min(jnp.where(offs9c > pb, offs9c,
                               float(N_ASSIGN)), axis=0, keepdims=True)
    hi_base = jnp.minimum(seg_hi, pb + 256.0)
    lo_base = pb
    r_base = iota32
    k_base = 16.0 * r_base

    ob = offs9[:, 1:8]
    obi = ob.astype(_i32)
    onx = offs9[:, 2:9]
    rb = jnp.minimum(obi // 256, N_TILES - 1)
    tile_end = (obi // 256) * 256 + 256
    hi_b = jnp.minimum(onx, tile_end.astype(_f32))
    hi_b = jnp.where(obi % 256 == 0, ob, hi_b)
    e_b = (lax.broadcasted_iota(_i32, (1, 7), 1) + 1).astype(_f32)
    k_b = (rb * 16).astype(_f32) + e_b

    pad = jnp.full((1, 1), 1.0, _f32)
    e_all = jnp.concatenate([e_base, e_b, pad * 7], axis=1)
    r_all = jnp.concatenate([r_base, rb.astype(_f32), pad * 31], axis=1)
    lo_all = jnp.concatenate([lo_base, ob, pad * N_ASSIGN], axis=1)
    hi_all = jnp.concatenate([hi_base, hi_b, pad * N_ASSIGN], axis=1)
    k_all = jnp.concatenate([k_base, k_b, pad * (31 * 16 + 15)], axis=1)

    k_col = _tr(k_all, N_ITEMS)
    rank_col = jnp.sum((k_all < k_col).astype(_f32), axis=1,
                       keepdims=True)
    slot = lax.broadcasted_iota(_i32, (N_ITEMS, N_ITEMS), 1).astype(_f32)
    perm = (rank_col == slot).astype(_f32)
    def srt(vrow):
        return lax.dot_general(perm, _tr(vrow, N_ITEMS),
                               (((0,), (0,)), ((), ())),
                               preferred_element_type=_f32,
                               precision=lax.Precision.HIGHEST)
    e_s = srt(e_all)
    r_s = srt(r_all)
    lo_s = srt(lo_all)
    hi_s = srt(hi_all)
    items_ref[...] = jnp.concatenate(
        [e_s, r_s, lo_s, hi_s], axis=1).astype(_i32)


def _route_call(x, wg, bg):
    return pl.pallas_call(
        _route_body,
        out_shape=[
            jax.ShapeDtypeStruct((2, N_TOK), _i32),
            jax.ShapeDtypeStruct((2, N_TOK), _f32),
            jax.ShapeDtypeStruct((N_ITEMS, 4), _i32),
        ],
    )(x, wg, bg.reshape(1, N_EXP))


def _sc_gather_body(x_hbm, pos_hbm, xs_hbm, pos_v, tok_v, rows_v, sem):
    nc = 2
    wid = lax.axis_index("s") * nc + lax.axis_index("c")
    pltpu.sync_copy(pos_hbm, pos_v)

    lane = lax.iota(_i32, 16)

    def scat(nb, _):
        idx16 = pos_v[pl.ds(nb * 16, 16)]
        val = nb * 16 + lane
        val = jnp.where(val >= N_TOK, val - N_TOK, val)
        plsc.store_scatter(tok_v, [idx16], val)
        return 0

    lax.fori_loop(0, N_ASSIGN // 16, scat, 0)

    base = wid * (N_ASSIGN // 32)
    for c in range(4):
        idx = tok_v.at[pl.ds(base + c * 64, 64)]
        pltpu.async_copy(x_hbm.at[idx], rows_v, sem).wait()
        pltpu.sync_copy(rows_v, xs_hbm.at[pl.ds(base + c * 64, 64)])


def _sc_gather_call(x, pos_flat):
    mesh = plsc.VectorSubcoreMesh(core_axis_name="c", subcore_axis_name="s")
    f = functools.partial(
        pl.kernel,
        mesh=mesh,
        out_type=jax.ShapeDtypeStruct((N_ASSIGN, D_MODEL), _f32),
        scratch_types=[
            pltpu.VMEM((N_ASSIGN,), _i32),
            pltpu.VMEM((N_ASSIGN,), _i32),
            pltpu.VMEM((64, D_MODEL), _f32),
            pltpu.SemaphoreType.DMA,
        ],
        compiler_params=pltpu.CompilerParams(needs_layout_passes=False),
    )(_sc_gather_body)
    return f(x, pos_flat)


def _ffn_body(items_ref, xs_ref, w1_ref, b1_ref, w2_ref, b2_ref, y_ref):
    i = pl.program_id(0)
    f = pl.program_id(1)
    r = items_ref[i, 1]
    lo = items_ref[i, 2]
    hi = items_ref[i, 3]
    prev_r = items_ref[jnp.maximum(i - 1, 0), 1]
    first_for_tile = jnp.logical_or(i == 0, r != prev_r)
    init = jnp.logical_and(first_for_tile, f == 0)

    x_t = xs_ref[...].astype(jnp.bfloat16)
    w1t = w1_ref[0].astype(jnp.bfloat16)
    h = lax.dot_general(x_t, w1t, (((1,), (1,)), ((), ())),
                        preferred_element_type=_f32)
    h = h + b1_ref[0, 0, pl.ds(f * FF_CHUNK, FF_CHUNK)][None, :]
    h = 0.5 * h * (1.0 + lax.erf(h * 0.7071067811865476))

    w2t = w2_ref[0].astype(jnp.bfloat16)
    yc = lax.dot_general(h.astype(jnp.bfloat16), w2t,
                         (((1,), (1,)), ((), ())),
                         preferred_element_type=_f32)
    add_b2 = jnp.where(f == 0, 1.0, 0.0)
    yc = yc + add_b2 * b2_ref[0, 0][None, :]

    rows = TILE * r + lax.broadcasted_iota(_i32, (TILE, 1), 0)
    seg = jnp.logical_and(rows >= lo, rows < hi)
    contrib = jnp.where(seg, yc, 0.0)

    @pl.when(init)
    def _():
        y_ref[...] = contrib

    @pl.when(jnp.logical_not(init))
    def _():
        y_ref[...] = y_ref[...] + contrib


def _ffn_call(items, xs, w1, b1, w2, b2):
    grid_spec = pltpu.PrefetchScalarGridSpec(
        num_scalar_prefetch=1,
        grid=(N_ITEMS, N_FF),
        in_specs=[
            pl.BlockSpec((TILE, D_MODEL), lambda i, f, it: (it[i, 1], 0)),
            pl.BlockSpec((1, FF_CHUNK, D_MODEL),
                         lambda i, f, it: (it[i, 0], f, 0)),
            pl.BlockSpec((1, 1, D_FF), lambda i, f, it: (it[i, 0], 0, 0)),
            pl.BlockSpec((1, D_MODEL, FF_CHUNK),
                         lambda i, f, it: (it[i, 0], 0, f)),
            pl.BlockSpec((1, 1, D_MODEL), lambda i, f, it: (it[i, 0], 0, 0)),
        ],
        out_specs=pl.BlockSpec((TILE, D_MODEL), lambda i, f, it: (it[i, 1], 0)),
    )
    return pl.pallas_call(
        _ffn_body,
        grid_spec=grid_spec,
        out_shape=jax.ShapeDtypeStruct((N_ASSIGN, D_MODEL), _f32),
        compiler_params=pltpu.CompilerParams(
            dimension_semantics=("arbitrary", "arbitrary")),
    )(items, xs, w1, b1.reshape(N_EXP, 1, D_FF), w2,
      b2.reshape(N_EXP, 1, D_MODEL))


def _sc_combine_body(y_hbm, p0_hbm, p1_hbm, w0_hbm, w1_hbm, out_hbm,
                     idx0_v, idx1_v, w0_v, w1_v, buf0, buf1, out_v, sem):
    nc = 2
    wid = lax.axis_index("s") * nc + lax.axis_index("c")
    tbase = wid * (N_TOK // 32)
    zero16 = jnp.zeros((16,), _i32)

    for c in range(4):
        base = tbase + c * 32
        pltpu.sync_copy(p0_hbm.at[pl.ds(base, 32)], idx0_v)
        pltpu.sync_copy(p1_hbm.at[pl.ds(base, 32)], idx1_v)
        pltpu.sync_copy(w0_hbm.at[pl.ds(base, 32)], w0_v)
        pltpu.sync_copy(w1_hbm.at[pl.ds(base, 32)], w1_v)
        cp0 = pltpu.async_copy(y_hbm.at[idx0_v], buf0, sem)
        cp1 = pltpu.async_copy(y_hbm.at[idx1_v], buf1, sem)
        cp0.wait()
        cp1.wait()

        def tok(t, _):
            wa = plsc.load_gather(w0_v, [zero16 + t])
            wb = plsc.load_gather(w1_v, [zero16 + t])
            for d in range(D_MODEL // 16):
                a = buf0[t, pl.ds(d * 16, 16)]
                b = buf1[t, pl.ds(d * 16, 16)]
                out_v[t, pl.ds(d * 16, 16)] = wa * a + wb * b
            return 0

        lax.fori_loop(0, 32, tok, 0)
        pltpu.sync_copy(out_v, out_hbm.at[pl.ds(base, 32)])


def _sc_combine_call(y, p0, p1, w0, w1):
    mesh = plsc.VectorSubcoreMesh(core_axis_name="c", subcore_axis_name="s")
    f = functools.partial(
        pl.kernel,
        mesh=mesh,
        out_type=jax.ShapeDtypeStruct((N_TOK, D_MODEL), _f32),
        scratch_types=[
            pltpu.VMEM((32,), _i32),
            pltpu.VMEM((32,), _i32),
            pltpu.VMEM((32,), _f32),
            pltpu.VMEM((32,), _f32),
            pltpu.VMEM((32, D_MODEL), _f32),
            pltpu.VMEM((32, D_MODEL), _f32),
            pltpu.VMEM((32, D_MODEL), _f32),
            pltpu.SemaphoreType.DMA,
        ],
        compiler_params=pltpu.CompilerParams(needs_layout_passes=False),
    )(_sc_combine_body)
    return f(y, p0, p1, w0, w1)


def kernel(x, Wg, bg, W1, b1, W2, b2):
    pos, w01, items = _route_call(x, Wg, bg)
    xs = _sc_gather_call(x, pos.reshape(N_ASSIGN))
    y = _ffn_call(items, xs, W1, b1, W2, b2)
    return _sc_combine_call(y, pos[0], pos[1], w01[0], w01[1])


# speedup vs baseline: 1.1958x; 1.1958x over previous
import functools

import jax
import jax.numpy as jnp
from jax import lax
from jax.experimental import pallas as pl
from jax.experimental.pallas import tpu as pltpu
from jax.experimental.pallas import tpu_sc as plsc

D_MODEL = 1024
N_EXP = 8
N_TOK = 4096
D_FF = 4096
N_ASSIGN = 2 * N_TOK
TILE = 256
N_TILES = N_ASSIGN // TILE
N_ITEMS = 40
FF_CHUNK = 2048
N_FF = D_FF // FF_CHUNK

_f32 = jnp.float32
_i32 = jnp.int32


def _tr(v, n):
    r = lax.broadcasted_iota(_i32, (n, n), 0)
    c = lax.broadcasted_iota(_i32, (n, n), 1)
    ident = (r == c).astype(v.dtype)
    return lax.dot_general(ident, v, (((1,), (1,)), ((), ())),
                           preferred_element_type=v.dtype,
                           precision=lax.Precision.HIGHEST)


def _route_body(x_ref, wg_ref, bg_ref, pos_ref, w01_ref, items_ref):
    x = x_ref[...]
    wg = wg_ref[...]
    logits = lax.dot_general(x, wg, (((1,), (1,)), ((), ())),
                             preferred_element_type=_f32)
    logits = logits + bg_ref[...][0][None, :]

    m = jnp.max(logits, axis=1, keepdims=True)
    ex = jnp.exp(logits - m)
    w = ex / jnp.sum(ex, axis=1, keepdims=True)

    r8 = lax.broadcasted_iota(_i32, (N_EXP, N_EXP), 0)
    c8 = lax.broadcasted_iota(_i32, (N_EXP, N_EXP), 1)
    tri8 = (r8 <= c8).astype(_f32)
    m1 = jnp.max(logits, axis=1, keepdims=True)
    match1 = (logits == m1).astype(_f32)
    cum1 = lax.dot_general(match1, tri8, (((1,), (0,)), ((), ())),
                           preferred_element_type=_f32)
    first1 = match1 * (cum1 == 1.0).astype(_f32)
    l2 = jnp.where(first1 > 0, -jnp.inf, logits)
    m2 = jnp.max(l2, axis=1, keepdims=True)
    match2 = (l2 == m2).astype(_f32)
    cum2 = lax.dot_general(match2, tri8, (((1,), (0,)), ((), ())),
                           preferred_element_type=_f32)
    first2 = match2 * (cum2 == 1.0).astype(_f32)

    w0 = jnp.sum(first1 * w, axis=1)
    w1 = jnp.sum(first2 * w, axis=1)
    w01_ref[...] = jnp.stack([w0, w1], axis=0)

    oh = jnp.concatenate([first1, first2], axis=0)
    ch = 1024
    rch = lax.broadcasted_iota(_i32, (ch, ch), 0)
    cch = lax.broadcasted_iota(_i32, (ch, ch), 1)
    tri = (rch >= cch).astype(_f32)
    carry = jnp.zeros((1, N_EXP), _f32)
    cums = []
    for i in range(N_ASSIGN // ch):
        blk = oh[i * ch:(i + 1) * ch, :]
        cums.append(lax.dot_general(tri, blk, (((1,), (0,)), ((), ())),
                                    preferred_element_type=_f32) + carry)
        carry = carry + jnp.sum(blk, axis=0, keepdims=True)
    cum = jnp.concatenate(cums, axis=0)
    counts = carry

    ru = lax.broadcasted_iota(_i32, (N_EXP, N_EXP), 0)
    cu = lax.broadcasted_iota(_i32, (N_EXP, N_EXP), 1)
    strict = (ru < cu).astype(_f32)
    offs = lax.dot_general(counts, strict, (((1,), (0,)), ((), ())),
                           preferred_element_type=_f32,
                           precision=lax.Precision.HIGHEST)

    posf = jnp.sum(oh * (offs + cum - 1.0), axis=1)
    pos_ref[...] = posf.astype(_i32).reshape(2, N_TOK)

    offs9 = jnp.concatenate(
        [offs, jnp.full((1, 1), float(N_ASSIGN), _f32)], axis=1)
    offs9c = _tr(offs9, 9)

    iota32 = lax.broadcasted_iota(_i32, (1, N_TILES), 1).astype(_f32)
    pb = 256.0 * iota32
    le = (offs9c[0:8, :] <= pb).astype(_f32)
    e_base = jnp.sum(le, axis=0, keepdims=True) - 1.0
    seg_hi = jnp.min(jnp.where(offs9c > pb, offs9c,
                               float(N_ASSIGN)), axis=0, keepdims=True)
    hi_base = jnp.minimum(seg_hi, pb + 256.0)
    lo_base = pb
    r_base = iota32
    k_base = 16.0 * r_base

    ob = offs9[:, 1:8]
    obi = ob.astype(_i32)
    onx = offs9[:, 2:9]
    rb = jnp.minimum(obi // 256, N_TILES - 1)
    tile_end = (obi // 256) * 256 + 256
    hi_b = jnp.minimum(onx, tile_end.astype(_f32))
    hi_b = jnp.where(obi % 256 == 0, ob, hi_b)
    e_b = (lax.broadcasted_iota(_i32, (1, 7), 1) + 1).astype(_f32)
    k_b = (rb * 16).astype(_f32) + e_b

    pad = jnp.full((1, 1), 1.0, _f32)
    e_all = jnp.concatenate([e_base, e_b, pad * 7], axis=1)
    r_all = jnp.concatenate([r_base, rb.astype(_f32), pad * 31], axis=1)
    lo_all = jnp.concatenate([lo_base, ob, pad * N_ASSIGN], axis=1)
    hi_all = jnp.concatenate([hi_base, hi_b, pad * N_ASSIGN], axis=1)
    k_all = jnp.concatenate([k_base, k_b, pad * (31 * 16 + 15)], axis=1)

    k_col = _tr(k_all, N_ITEMS)
    rank_col = jnp.sum((k_all < k_col).astype(_f32), axis=1,
                       keepdims=True)
    slot = lax.broadcasted_iota(_i32, (N_ITEMS, N_ITEMS), 1).astype(_f32)
    perm = (rank_col == slot).astype(_f32)
    def srt(vrow):
        return lax.dot_general(perm, _tr(vrow, N_ITEMS),
                               (((0,), (0,)), ((), ())),
                               preferred_element_type=_f32,
                               precision=lax.Precision.HIGHEST)
    e_s = srt(e_all)
    r_s = srt(r_all)
    lo_s = srt(lo_all)
    hi_s = srt(hi_all)
    items_ref[...] = jnp.concatenate(
        [e_s, r_s, lo_s, hi_s], axis=1).astype(_i32)


def _route_call(x, wg, bg):
    return pl.pallas_call(
        _route_body,
        out_shape=[
            jax.ShapeDtypeStruct((2, N_TOK), _i32),
            jax.ShapeDtypeStruct((2, N_TOK), _f32),
            jax.ShapeDtypeStruct((N_ITEMS, 4), _i32),
        ],
    )(x, wg, bg.reshape(1, N_EXP))


def _sc_gather_body(x_hbm, pos_hbm, xs_hbm, pos_v, tok_v, rows_v, sem):
    nc = 2
    wid = lax.axis_index("s") * nc + lax.axis_index("c")
    pltpu.sync_copy(pos_hbm, pos_v)

    lane = lax.iota(_i32, 16)

    def scat(nb, _):
        idx16 = pos_v[pl.ds(nb * 16, 16)]
        val = nb * 16 + lane
        val = jnp.where(val >= N_TOK, val - N_TOK, val)
        plsc.store_scatter(tok_v, [idx16], val)
        return 0

    lax.fori_loop(0, N_ASSIGN // 16, scat, 0)

    base = wid * (N_ASSIGN // 32)
    for c in range(4):
        idx = tok_v.at[pl.ds(base + c * 64, 64)]
        pltpu.async_copy(x_hbm.at[idx], rows_v, sem).wait()
        pltpu.sync_copy(rows_v, xs_hbm.at[pl.ds(base + c * 64, 64)])


def _sc_gather_call(x, pos_flat):
    mesh = plsc.VectorSubcoreMesh(core_axis_name="c", subcore_axis_name="s")
    f = functools.partial(
        pl.kernel,
        mesh=mesh,
        out_type=jax.ShapeDtypeStruct((N_ASSIGN, D_MODEL), _f32),
        scratch_types=[
            pltpu.VMEM((N_ASSIGN,), _i32),
            pltpu.VMEM((N_ASSIGN,), _i32),
            pltpu.VMEM((64, D_MODEL), _f32),
            pltpu.SemaphoreType.DMA,
        ],
        compiler_params=pltpu.CompilerParams(needs_layout_passes=False),
    )(_sc_gather_body)
    return f(x, pos_flat)


def _ffn_body(items_ref, xs_ref, w1_ref, b1_ref, w2_ref, b2_ref, y_ref):
    f = pl.program_id(0)
    i = pl.program_id(1)
    r = items_ref[i, 1]
    lo = items_ref[i, 2]
    hi = items_ref[i, 3]
    prev_r = items_ref[jnp.maximum(i - 1, 0), 1]
    init = jnp.logical_or(i == 0, r != prev_r)

    x_t = xs_ref[...].astype(jnp.bfloat16)
    w1t = w1_ref[0].astype(jnp.bfloat16)
    h = lax.dot_general(x_t, w1t, (((1,), (1,)), ((), ())),
                        preferred_element_type=_f32)
    h = h + b1_ref[0, 0][None, :]
    h = 0.5 * h * (1.0 + lax.erf(h * 0.7071067811865476))

    w2t = w2_ref[0].astype(jnp.bfloat16)
    yc = lax.dot_general(h.astype(jnp.bfloat16), w2t,
                         (((1,), (1,)), ((), ())),
                         preferred_element_type=_f32)
    add_b2 = jnp.where(f == 0, 1.0, 0.0)
    yc = yc + add_b2 * b2_ref[0, 0][None, :]

    rows = TILE * r + lax.broadcasted_iota(_i32, (TILE, 1), 0)
    seg = jnp.logical_and(rows >= lo, rows < hi)
    contrib = jnp.where(seg, yc, 0.0)

    @pl.when(init)
    def _():
        y_ref[...] = contrib[None]

    @pl.when(jnp.logical_not(init))
    def _():
        y_ref[...] = y_ref[...] + contrib[None]


def _ffn_call(items, xs, w1, b1, w2, b2):
    grid_spec = pltpu.PrefetchScalarGridSpec(
        num_scalar_prefetch=1,
        grid=(N_FF, N_ITEMS),
        in_specs=[
            pl.BlockSpec((TILE, D_MODEL), lambda f, i, it: (it[i, 1], 0)),
            pl.BlockSpec((1, FF_CHUNK, D_MODEL),
                         lambda f, i, it: (it[i, 0], f, 0)),
            pl.BlockSpec((1, 1, FF_CHUNK), lambda f, i, it: (it[i, 0], 0, f)),
            pl.BlockSpec((1, D_MODEL, FF_CHUNK),
                         lambda f, i, it: (it[i, 0], 0, f)),
            pl.BlockSpec((1, 1, D_MODEL), lambda f, i, it: (it[i, 0], 0, 0)),
        ],
        out_specs=pl.BlockSpec((1, TILE, D_MODEL),
                               lambda f, i, it: (f, it[i, 1], 0)),
    )
    return pl.pallas_call(
        _ffn_body,
        grid_spec=grid_spec,
        out_shape=jax.ShapeDtypeStruct((N_FF, N_ASSIGN, D_MODEL), _f32),
        compiler_params=pltpu.CompilerParams(
            dimension_semantics=("arbitrary", "arbitrary")),
    )(items, xs, w1, b1.reshape(N_EXP, 1, D_FF), w2,
      b2.reshape(N_EXP, 1, D_MODEL))


def _sc_combine_body(y_hbm, p0_hbm, p1_hbm, w0_hbm, w1_hbm, out_hbm,
                     idx0_v, idx1_v, w0_v, w1_v, buf0, buf1, out_v, sem):
    nc = 2
    wid = lax.axis_index("s") * nc + lax.axis_index("c")
    tbase = wid * (N_TOK // 32)
    zero16 = jnp.zeros((16,), _i32)

    for c in range(4):
        base = tbase + c * 32
        pltpu.sync_copy(p0_hbm.at[pl.ds(base, 32)], idx0_v)
        pltpu.sync_copy(p1_hbm.at[pl.ds(base, 32)], idx1_v)
        pltpu.sync_copy(w0_hbm.at[pl.ds(base, 32)], w0_v)
        pltpu.sync_copy(w1_hbm.at[pl.ds(base, 32)], w1_v)

        for half in range(N_FF):
            if half:
                for q in range(2):
                    sl = pl.ds(q * 16, 16)
                    idx0_v[sl] = idx0_v[sl] + N_ASSIGN
                    idx1_v[sl] = idx1_v[sl] + N_ASSIGN
            cp0 = pltpu.async_copy(y_hbm.at[idx0_v], buf0, sem)
            cp1 = pltpu.async_copy(y_hbm.at[idx1_v], buf1, sem)
            cp0.wait()
            cp1.wait()

            if half == 0:
                def tok0(t, _):
                    wa = plsc.load_gather(w0_v, [zero16 + t])
                    wb = plsc.load_gather(w1_v, [zero16 + t])
                    for d in range(D_MODEL // 16):
                        sl = pl.ds(d * 16, 16)
                        out_v[t, sl] = wa * buf0[t, sl] + wb * buf1[t, sl]
                    return 0
                lax.fori_loop(0, 32, tok0, 0)
            else:
                def tok1(t, _):
                    wa = plsc.load_gather(w0_v, [zero16 + t])
                    wb = plsc.load_gather(w1_v, [zero16 + t])
                    for d in range(D_MODEL // 16):
                        sl = pl.ds(d * 16, 16)
                        out_v[t, sl] = (out_v[t, sl]
                                        + wa * buf0[t, sl] + wb * buf1[t, sl])
                    return 0
                lax.fori_loop(0, 32, tok1, 0)
        pltpu.sync_copy(out_v, out_hbm.at[pl.ds(base, 32)])


def _sc_combine_call(y, p0, p1, w0, w1):
    y = y.reshape(N_FF * N_ASSIGN, D_MODEL)
    mesh = plsc.VectorSubcoreMesh(core_axis_name="c", subcore_axis_name="s")
    f = functools.partial(
        pl.kernel,
        mesh=mesh,
        out_type=jax.ShapeDtypeStruct((N_TOK, D_MODEL), _f32),
        scratch_types=[
            pltpu.VMEM((32,), _i32),
            pltpu.VMEM((32,), _i32),
            pltpu.VMEM((32,), _f32),
            pltpu.VMEM((32,), _f32),
            pltpu.VMEM((32, D_MODEL), _f32),
            pltpu.VMEM((32, D_MODEL), _f32),
            pltpu.VMEM((32, D_MODEL), _f32),
            pltpu.SemaphoreType.DMA,
        ],
        compiler_params=pltpu.CompilerParams(needs_layout_passes=False),
    )(_sc_combine_body)
    return f(y, p0, p1, w0, w1)


def kernel(x, Wg, bg, W1, b1, W2, b2):
    pos, w01, items = _route_call(x, Wg, bg)
    xs = _sc_gather_call(x, pos.reshape(N_ASSIGN))
    y = _ffn_call(items, xs, W1, b1, W2, b2)
    return _sc_combine_call(y, pos[0], pos[1], w01[0], w01[1])
